# baseline (speedup 1.0000x reference)
import jax
import jax.numpy as jnp
from jax import lax
from jax.experimental import pallas as pl
from jax.experimental.pallas import tpu as pltpu
from jax.experimental.pallas import tpu_sc as plsc

NUM_EXPERTS = 8
TOP_K = 2
HIDDEN = 1024
BT = 2048
NC, NS, L = 2, 16, 16
NW = NC * NS
SC_T = 4096
TPW = SC_T // NW
GROUPS = TPW // L
T_REST = 32768 - SC_T


def _logits_block(x_ref, w_ref, logits_ref):
    logits_ref[...] = jax.lax.dot_general(
        w_ref[...], x_ref[...],
        dimension_numbers=(((1,), (1,)), ((), ())),
        preferred_element_type=jnp.float32,
    )


def _tc_logits_head(x, W):
    nblk = SC_T // BT
    return pl.pallas_call(
        _logits_block,
        grid=(nblk,),
        in_specs=[
            pl.BlockSpec((BT, HIDDEN), lambda i: (i, 0)),
            pl.BlockSpec((NUM_EXPERTS, HIDDEN), lambda i: (0, 0)),
        ],
        out_specs=pl.BlockSpec((NUM_EXPERTS, BT), lambda i: (0, i)),
        out_shape=jax.ShapeDtypeStruct((NUM_EXPERTS, SC_T), jnp.float32),
    )(x, W)


def _fused_block(x_ref, w_ref, logits_ref, aff_ref, idx_ref):
    x = x_ref[...]
    w = w_ref[...]
    logits = jax.lax.dot_general(
        w, x,
        dimension_numbers=(((1,), (1,)), ((), ())),
        preferred_element_type=jnp.float32,
    )
    m = jnp.max(logits, axis=0, keepdims=True)
    e = jnp.exp(logits - m)
    s = jnp.sum(e, axis=0, keepdims=True)
    aff = e * (1.0 / s)

    iota = jax.lax.broadcasted_iota(jnp.int32, aff.shape, 0)
    big = jnp.int32(NUM_EXPERTS)
    v1 = jnp.max(aff, axis=0, keepdims=True)
    idx1 = jnp.min(jnp.where(aff == v1, iota, big), axis=0, keepdims=True)
    aff2 = jnp.where(iota == idx1, -1.0, aff)
    v2 = jnp.max(aff2, axis=0, keepdims=True)
    idx2 = jnp.min(jnp.where(aff2 == v2, iota, big), axis=0, keepdims=True)

    logits_ref[...] = logits
    aff_ref[...] = aff
    idx_ref[...] = jnp.concatenate([idx1, idx2], axis=0)


def _tc_fused_rest(x, W, T):
    nblk = T_REST // BT
    blk0 = SC_T // BT
    return pl.pallas_call(
        _fused_block,
        grid=(nblk,),
        in_specs=[
            pl.BlockSpec((BT, HIDDEN), lambda i: (blk0 + i, 0)),
            pl.BlockSpec((NUM_EXPERTS, HIDDEN), lambda i: (0, 0)),
        ],
        out_specs=[
            pl.BlockSpec((NUM_EXPERTS, BT), lambda i: (0, blk0 + i)),
            pl.BlockSpec((NUM_EXPERTS, BT), lambda i: (0, blk0 + i)),
            pl.BlockSpec((TOP_K, BT), lambda i: (0, blk0 + i)),
        ],
        out_shape=[
            jax.ShapeDtypeStruct((NUM_EXPERTS, T), jnp.float32),
            jax.ShapeDtypeStruct((NUM_EXPERTS, T), jnp.float32),
            jax.ShapeDtypeStruct((TOP_K, T), jnp.int32),
        ],
    )(x, W)


def _sc_route_body(logits_hbm, aff_out, idx_out, logits_v, aff_v, idx_v):
    wid = lax.axis_index("s") * NC + lax.axis_index("c")
    tok0 = wid * TPW

    pltpu.sync_copy(logits_hbm.at[:, pl.ds(tok0, TPW)], logits_v)

    def gbody(g, _):
        off = g * L
        l = [logits_v[e, pl.ds(off, L)] for e in range(NUM_EXPERTS)]
        m = l[0]
        for e in range(1, NUM_EXPERTS):
            m = jnp.maximum(m, l[e])
        ex = [jnp.exp(v - m) for v in l]
        s = ex[0]
        for e in range(1, NUM_EXPERTS):
            s = s + ex[e]
        r = 1.0 / s
        a = [v * r for v in ex]
        for e in range(NUM_EXPERTS):
            aff_v[e, pl.ds(off, L)] = a[e]
        best = a[0]
        bidx = jnp.zeros((L,), jnp.int32)
        second = jnp.full((L,), -1.0, jnp.float32)
        sidx = jnp.zeros((L,), jnp.int32)
        for e in range(1, NUM_EXPERTS):
            esp = jnp.full((L,), e, jnp.int32)
            gt_best = a[e] > best
            gt_sec = a[e] > second
            second = jnp.where(gt_best, best, jnp.where(gt_sec, a[e], second))
            sidx = jnp.where(gt_best, bidx, jnp.where(gt_sec, esp, sidx))
            best = jnp.where(gt_best, a[e], best)
            bidx = jnp.where(gt_best, esp, bidx)
        idx_v[0, pl.ds(off, L)] = bidx
        idx_v[1, pl.ds(off, L)] = sidx
        return 0

    lax.fori_loop(0, GROUPS, gbody, 0)

    pltpu.sync_copy(aff_v, aff_out.at[:, pl.ds(tok0, TPW)])
    pltpu.sync_copy(idx_v, idx_out.at[:, pl.ds(tok0, TPW)])


def _sc_route(logits_head):
    mesh = plsc.VectorSubcoreMesh(core_axis_name="c", subcore_axis_name="s")
    k = pl.kernel(
        _sc_route_body,
        out_type=[
            jax.ShapeDtypeStruct((NUM_EXPERTS, SC_T), jnp.float32),
            jax.ShapeDtypeStruct((TOP_K, SC_T), jnp.int32),
        ],
        mesh=mesh,
        compiler_params=pltpu.CompilerParams(skip_device_barrier=True),
        scratch_types=[
            pltpu.VMEM((NUM_EXPERTS, TPW), jnp.float32),
            pltpu.VMEM((NUM_EXPERTS, TPW), jnp.float32),
            pltpu.VMEM((TOP_K, TPW), jnp.int32),
        ],
    )
    return k(logits_head)


@jax.jit
def _router(x, W):
    T = x.shape[0]
    logits_head = _tc_logits_head(x, W)
    aff_head, idx_head = _sc_route(logits_head)
    logits_w, aff_w, idx_w = _tc_fused_rest(x, W, T)
    logits_w = lax.dynamic_update_slice(logits_w, logits_head, (0, 0))
    aff_w = lax.dynamic_update_slice(aff_w, aff_head, (0, 0))
    idx_w = lax.dynamic_update_slice(idx_w, idx_head, (0, 0))
    return logits_w.T, aff_w.T, idx_w.T


def kernel(hidden_states, W):
    B, S, H = hidden_states.shape
    x = hidden_states.reshape(B * S, H)
    return _router(x, W)


# speedup vs baseline: 1.4819x; 1.4819x over previous
import jax
import jax.numpy as jnp
from jax.experimental import pallas as pl

NUM_EXPERTS = 8
TOP_K = 2
HIDDEN = 1024
BT = 2048


def _router_block(x_ref, w_ref, logits_ref, aff_ref, idx_ref):
    x = x_ref[...]
    w = w_ref[...]
    logits = jax.lax.dot_general(
        w, x,
        dimension_numbers=(((1,), (1,)), ((), ())),
        preferred_element_type=jnp.float32,
    )
    m = jnp.max(logits, axis=0, keepdims=True)
    e = jnp.exp(logits - m)
    s = jnp.sum(e, axis=0, keepdims=True)
    aff = e * (1.0 / s)

    iota = jax.lax.broadcasted_iota(jnp.int32, aff.shape, 0)
    big = jnp.int32(NUM_EXPERTS)
    v1 = jnp.max(aff, axis=0, keepdims=True)
    idx1 = jnp.min(jnp.where(aff == v1, iota, big), axis=0, keepdims=True)
    aff2 = jnp.where(iota == idx1, -1.0, aff)
    v2 = jnp.max(aff2, axis=0, keepdims=True)
    idx2 = jnp.min(jnp.where(aff2 == v2, iota, big), axis=0, keepdims=True)

    logits_ref[...] = logits
    aff_ref[...] = aff
    idx_ref[...] = jnp.concatenate([idx1, idx2], axis=0)


@jax.jit
def _router(x, W):
    T = x.shape[0]
    nblk = T // BT
    logits_w, aff_w, idx_w = pl.pallas_call(
        _router_block,
        grid=(nblk,),
        in_specs=[
            pl.BlockSpec((BT, HIDDEN), lambda i: (i, 0)),
            pl.BlockSpec((NUM_EXPERTS, HIDDEN), lambda i: (0, 0)),
        ],
        out_specs=[
            pl.BlockSpec((NUM_EXPERTS, BT), lambda i: (0, i)),
            pl.BlockSpec((NUM_EXPERTS, BT), lambda i: (0, i)),
            pl.BlockSpec((TOP_K, BT), lambda i: (0, i)),
        ],
        out_shape=[
            jax.ShapeDtypeStruct((NUM_EXPERTS, T), jnp.float32),
            jax.ShapeDtypeStruct((NUM_EXPERTS, T), jnp.float32),
            jax.ShapeDtypeStruct((TOP_K, T), jnp.int32),
        ],
    )(x, W)
    return logits_w.T, aff_w.T, idx_w.T


def kernel(hidden_states, W):
    B, S, H = hidden_states.shape
    x = hidden_states.reshape(B * S, H)
    return _router(x, W)
